# baseline (speedup 1.0000x reference)
import jax
import jax.numpy as jnp
from jax import lax
from jax.experimental import pallas as pl
from jax.experimental.pallas import tpu as pltpu
from jax.experimental.pallas import tpu_sc as plsc

_NC = 2
_NS = 16
_NW = _NC * _NS
_L = 16

_B = 800


def _softplus_terms(d2, bias):
    d2c = jnp.maximum(d2, 1e-24)
    bits = plsc.bitcast(d2c, jnp.int32)
    r = plsc.bitcast(jnp.int32(0x5F3759DF) - (bits >> 1), jnp.float32)
    for _ in range(3):
        r = r * (1.5 - 0.5 * d2c * r * r)
    dist = d2c * r
    x = 10.0 * (bias - (dist + 1e-8))
    u = jnp.exp(-jnp.abs(x))
    s = u / (2.0 + u)
    s2 = s * s
    lnv = 2.0 * s * (1.0 + s2 * (1.0 / 3.0 + s2 * (0.2 + s2 * (1.0 / 7.0))))
    return (jnp.maximum(x, 0.0) + lnv) * 0.1


def _sc_likelihood(z_hbm, w_hbm, ai_hbm, aj_hbm, bias_hbm, out_hbm,
                   ii0, jj0, zr0, wr0, ii1, jj1, zr1, wr1, ii2, jj2, zr2, wr2,
                   accv, biasv, sin0, sin1, sin2, six):
    E = ai_hbm.shape[0]
    epw = E // _NW
    nblk = epw // _B
    assert nblk % 3 == 2
    bufs = ((ii0, jj0, zr0, wr0, sin0), (ii1, jj1, zr1, wr1, sin1),
            (ii2, jj2, zr2, wr2, sin2))

    wid = lax.axis_index("s") * _NC + lax.axis_index("c")
    base = wid * epw

    lane = lax.iota(jnp.int32, _L)
    pltpu.sync_copy(bias_hbm, biasv)
    bias = biasv[...]

    def fire_gathers(buf):
        ii, jj, zr, wr, sin = buf
        pltpu.async_copy(z_hbm.at[ii], zr, sin)
        pltpu.async_copy(w_hbm.at[jj], wr, sin)

    def drain_in(buf):
        ii, jj, zr, wr, sin = buf
        pltpu.make_async_copy(z_hbm.at[ii], zr, sin).wait()
        pltpu.make_async_copy(w_hbm.at[jj], wr, sin).wait()

    def compute(buf, acc_in):
        ii, jj, zr, wr, sin = buf

        def group_body(g, acc):
            e16 = g * _L + lane
            d2 = jnp.zeros((_L,), jnp.float32)
            for d in range(16):
                dvec = jnp.bitwise_and(lane + d, _L - 1)
                zc = plsc.load_gather(zr, [e16, dvec])
                wc = plsc.load_gather(wr, [e16, dvec])
                t = zc - wc
                d2 = d2 + t * t
            return acc + _softplus_terms(d2, bias)

        return lax.fori_loop(0, _B // _L, group_body, acc_in, unroll=False)

    def fire_idx(b, buf):
        ii, jj, zr, wr, sin = buf
        off = base + b * _B
        pltpu.async_copy(ai_hbm.at[pl.ds(off, _B)], ii, six)
        pltpu.async_copy(aj_hbm.at[pl.ds(off, _B)], jj, six)

    def wait_idx(buf):
        ii, jj, zr, wr, sin = buf
        pltpu.make_async_copy(ai_hbm.at[pl.ds(base, _B)], ii, six).wait()
        pltpu.make_async_copy(aj_hbm.at[pl.ds(base, _B)], jj, six).wait()

    fire_idx(0, bufs[0])
    wait_idx(bufs[0])
    fire_gathers(bufs[0])
    fire_idx(1, bufs[1])
    wait_idx(bufs[1])
    fire_gathers(bufs[1])
    fire_idx(2, bufs[2])
    acc0 = jnp.zeros((_L,), jnp.float32)

    def super_body(sstep, acc):
        for third in range(3):
            b = 3 * sstep + third
            buf_b = bufs[third]
            buf_n = bufs[(third + 2) % 3]
            drain_in(buf_b)

            @pl.when(b + 3 < nblk)
            def _():
                fire_idx(b + 3, buf_b)

            wait_idx(buf_n)
            fire_gathers(buf_n)
            acc = compute(buf_b, acc)
        return acc

    acc = lax.fori_loop(0, nblk // 3, super_body, acc0, unroll=False)

    for b in (nblk - 2, nblk - 1):
        drain_in(bufs[b % 3])
        acc = compute(bufs[b % 3], acc)

    accv[...] = acc
    pltpu.sync_copy(accv, out_hbm.at[wid])


def _tc_reduce_body(part_ref, out_ref):
    out_ref[0, 0] = jnp.sum(part_ref[...])


def kernel(latent_z, latent_w, bias, analytical_i, analytical_j):
    sc_call = pl.kernel(
        _sc_likelihood,
        out_type=jax.ShapeDtypeStruct((_NW, _L), jnp.float32),
        mesh=plsc.VectorSubcoreMesh(core_axis_name="c", subcore_axis_name="s"),
        scratch_types=[
            pltpu.VMEM((_B,), jnp.int32),
            pltpu.VMEM((_B,), jnp.int32),
            pltpu.VMEM((_B, 16), jnp.float32),
            pltpu.VMEM((_B, 16), jnp.float32),
            pltpu.VMEM((_B,), jnp.int32),
            pltpu.VMEM((_B,), jnp.int32),
            pltpu.VMEM((_B, 16), jnp.float32),
            pltpu.VMEM((_B, 16), jnp.float32),
            pltpu.VMEM((_B,), jnp.int32),
            pltpu.VMEM((_B,), jnp.int32),
            pltpu.VMEM((_B, 16), jnp.float32),
            pltpu.VMEM((_B, 16), jnp.float32),
            pltpu.VMEM((_L,), jnp.float32),
            pltpu.VMEM((_L,), jnp.float32),
            pltpu.SemaphoreType.DMA,
            pltpu.SemaphoreType.DMA,
            pltpu.SemaphoreType.DMA,
            pltpu.SemaphoreType.DMA,
        ],
        compiler_params=pltpu.CompilerParams(
            needs_layout_passes=False, use_tc_tiling_on_sc=False),
    )
    partials = sc_call(latent_z, latent_w, analytical_i, analytical_j,
                       jnp.broadcast_to(bias, (_L,)))

    out = pl.pallas_call(
        _tc_reduce_body,
        out_shape=jax.ShapeDtypeStruct((1, 1), jnp.float32),
        out_specs=pl.BlockSpec(memory_space=pltpu.SMEM),
    )(partials)
    return out[0, 0]


# speedup vs baseline: 1.0098x; 1.0098x over previous
import jax
import jax.numpy as jnp
from jax import lax
from jax.experimental import pallas as pl
from jax.experimental.pallas import tpu as pltpu
from jax.experimental.pallas import tpu_sc as plsc

_NC = 2
_NS = 16
_NW = _NC * _NS
_L = 16

_B = 800


def _softplus_terms(d2, bias):
    d2c = jnp.maximum(d2, 1e-24)
    bits = plsc.bitcast(d2c, jnp.int32)
    r = plsc.bitcast(jnp.int32(0x5F3759DF) - (bits >> 1), jnp.float32)
    for _ in range(3):
        r = r * (1.5 - 0.5 * d2c * r * r)
    dist = d2c * r
    x = 10.0 * (bias - (dist + 1e-8))
    u = jnp.exp(-jnp.abs(x))
    s = u / (2.0 + u)
    s2 = s * s
    lnv = 2.0 * s * (1.0 + s2 * (1.0 / 3.0 + s2 * (0.2 + s2 * (1.0 / 7.0))))
    return (jnp.maximum(x, 0.0) + lnv) * 0.1


def _sc_likelihood(z_hbm, w_hbm, ai_hbm, aj_hbm, bias_hbm, out_hbm,
                   ii0, jj0, zr0, wr0, ii1, jj1, zr1, wr1, ii2, jj2, zr2, wr2,
                   accv, biasv, sin0, sin1, sin2, six):
    E = ai_hbm.shape[0]
    epw = E // _NW
    nblk = epw // _B
    assert nblk % 3 == 2
    bufs = ((ii0, jj0, zr0, wr0, sin0), (ii1, jj1, zr1, wr1, sin1),
            (ii2, jj2, zr2, wr2, sin2))

    wid = lax.axis_index("s") * _NC + lax.axis_index("c")
    base = wid * epw

    lane = lax.iota(jnp.int32, _L)
    pltpu.sync_copy(bias_hbm, biasv)
    bias = biasv[...]

    def fire_gathers(buf):
        ii, jj, zr, wr, sin = buf
        pltpu.async_copy(z_hbm.at[ii], zr, sin)
        pltpu.async_copy(w_hbm.at[jj], wr, sin)

    def drain_in(buf):
        ii, jj, zr, wr, sin = buf
        pltpu.make_async_copy(z_hbm.at[ii], zr, sin).wait()
        pltpu.make_async_copy(w_hbm.at[jj], wr, sin).wait()

    def compute(buf, acc_in):
        ii, jj, zr, wr, sin = buf

        def group_body(g, acc):
            e16 = g * _L + lane
            d2 = jnp.zeros((_L,), jnp.float32)
            for d in range(16):
                dvec = jnp.bitwise_and(lane + d, _L - 1)
                zc = plsc.load_gather(zr, [e16, dvec])
                wc = plsc.load_gather(wr, [e16, dvec])
                t = zc - wc
                d2 = d2 + t * t
            return acc + _softplus_terms(d2, bias)

        return lax.fori_loop(0, _B // _L, group_body, acc_in, unroll=False)

    def stage_and_fire(b, buf):
        ii, jj, zr, wr, sin = buf
        off = base + b * _B
        a = pltpu.async_copy(ai_hbm.at[pl.ds(off, _B)], ii, six)
        c = pltpu.async_copy(aj_hbm.at[pl.ds(off, _B)], jj, six)
        a.wait()
        c.wait()
        fire_gathers(buf)

    stage_and_fire(0, bufs[0])
    stage_and_fire(1, bufs[1])
    acc0 = jnp.zeros((_L,), jnp.float32)

    def super_body(sstep, acc):
        for third in range(3):
            b = 3 * sstep + third
            stage_and_fire(b + 2, bufs[(third + 2) % 3])
            drain_in(bufs[third])
            acc = compute(bufs[third], acc)
        return acc

    acc = lax.fori_loop(0, nblk // 3, super_body, acc0, unroll=False)

    for b in (nblk - 2, nblk - 1):
        drain_in(bufs[b % 3])
        acc = compute(bufs[b % 3], acc)

    accv[...] = acc
    pltpu.sync_copy(accv, out_hbm.at[wid])


def _tc_reduce_body(part_ref, out_ref):
    out_ref[0, 0] = jnp.sum(part_ref[...])


def kernel(latent_z, latent_w, bias, analytical_i, analytical_j):
    sc_call = pl.kernel(
        _sc_likelihood,
        out_type=jax.ShapeDtypeStruct((_NW, _L), jnp.float32),
        mesh=plsc.VectorSubcoreMesh(core_axis_name="c", subcore_axis_name="s"),
        scratch_types=[
            pltpu.VMEM((_B,), jnp.int32),
            pltpu.VMEM((_B,), jnp.int32),
            pltpu.VMEM((_B, 16), jnp.float32),
            pltpu.VMEM((_B, 16), jnp.float32),
            pltpu.VMEM((_B,), jnp.int32),
            pltpu.VMEM((_B,), jnp.int32),
            pltpu.VMEM((_B, 16), jnp.float32),
            pltpu.VMEM((_B, 16), jnp.float32),
            pltpu.VMEM((_B,), jnp.int32),
            pltpu.VMEM((_B,), jnp.int32),
            pltpu.VMEM((_B, 16), jnp.float32),
            pltpu.VMEM((_B, 16), jnp.float32),
            pltpu.VMEM((_L,), jnp.float32),
            pltpu.VMEM((_L,), jnp.float32),
            pltpu.SemaphoreType.DMA,
            pltpu.SemaphoreType.DMA,
            pltpu.SemaphoreType.DMA,
            pltpu.SemaphoreType.DMA,
        ],
        compiler_params=pltpu.CompilerParams(
            needs_layout_passes=False, use_tc_tiling_on_sc=False),
    )
    partials = sc_call(latent_z, latent_w, analytical_i, analytical_j,
                       jnp.broadcast_to(bias, (_L,)))

    out = pl.pallas_call(
        _tc_reduce_body,
        out_shape=jax.ShapeDtypeStruct((1, 1), jnp.float32),
        out_specs=pl.BlockSpec(memory_space=pltpu.SMEM),
    )(partials)
    return out[0, 0]


# speedup vs baseline: 1.0136x; 1.0038x over previous
import jax
import jax.numpy as jnp
from jax import lax
from jax.experimental import pallas as pl
from jax.experimental.pallas import tpu as pltpu
from jax.experimental.pallas import tpu_sc as plsc

_NC = 2
_NS = 16
_NW = _NC * _NS
_L = 16

_B = 800


def _softplus_terms(d2, bias):
    d2c = jnp.maximum(d2, 1e-24)
    bits = plsc.bitcast(d2c, jnp.int32)
    r = plsc.bitcast(jnp.int32(0x5F3759DF) - (bits >> 1), jnp.float32)
    for _ in range(3):
        r = r * (1.5 - 0.5 * d2c * r * r)
    dist = d2c * r
    x = 10.0 * (bias - (dist + 1e-8))
    u = jnp.exp(-jnp.abs(x))
    s = u / (2.0 + u)
    s2 = s * s
    lnv = 2.0 * s * (1.0 + s2 * (1.0 / 3.0 + s2 * (0.2 + s2 * (1.0 / 7.0))))
    return (jnp.maximum(x, 0.0) + lnv) * 0.1


def _sc_likelihood(z_hbm, w_hbm, ai_hbm, aj_hbm, bias_hbm, out_hbm,
                   ii0, jj0, zr0, wr0, ii1, jj1, zr1, wr1, ii2, jj2, zr2, wr2,
                   accv, biasv, sin0, sin1, sin2, six):
    E = ai_hbm.shape[0]
    epw = E // _NW
    nblk = epw // _B
    assert nblk % 3 == 2
    bufs = ((ii0, jj0, zr0, wr0, sin0), (ii1, jj1, zr1, wr1, sin1),
            (ii2, jj2, zr2, wr2, sin2))

    wid = lax.axis_index("s") * _NC + lax.axis_index("c")
    base = wid * epw

    lane = lax.iota(jnp.int32, _L)
    pltpu.sync_copy(bias_hbm, biasv)
    bias = biasv[...]

    def fire_gathers(buf):
        ii, jj, zr, wr, sin = buf
        pltpu.async_copy(z_hbm.at[ii], zr, sin)
        pltpu.async_copy(w_hbm.at[jj], wr, sin)

    def drain_in(buf):
        ii, jj, zr, wr, sin = buf
        pltpu.make_async_copy(z_hbm.at[ii], zr, sin).wait()
        pltpu.make_async_copy(w_hbm.at[jj], wr, sin).wait()

    def compute(buf, acc_in):
        ii, jj, zr, wr, sin = buf

        def group_body(g, acc):
            e16 = g * _L + lane
            d2 = jnp.zeros((_L,), jnp.float32)
            for d in range(16):
                dvec = jnp.bitwise_and(lane + d, _L - 1)
                zc = plsc.load_gather(zr, [e16, dvec])
                wc = plsc.load_gather(wr, [e16, dvec])
                t = zc - wc
                d2 = d2 + t * t
            return acc + _softplus_terms(d2, bias)

        return lax.fori_loop(0, _B // _L, group_body, acc_in, unroll=False)

    def stage_and_fire(b, buf):
        ii, jj, zr, wr, sin = buf
        off = base + b * _B
        a = pltpu.async_copy(ai_hbm.at[pl.ds(off, _B)], ii, six)
        c = pltpu.async_copy(aj_hbm.at[pl.ds(off, _B)], jj, six)
        a.wait()
        c.wait()
        fire_gathers(buf)

    stage_and_fire(0, bufs[0])
    stage_and_fire(1, bufs[1])
    acc0 = jnp.zeros((_L,), jnp.float32)

    def super_body(sstep, acc):
        for third in range(3):
            b = 3 * sstep + third
            stage_and_fire(b + 2, bufs[(third + 2) % 3])
            drain_in(bufs[third])
            acc = compute(bufs[third], acc)
        return acc

    acc = lax.fori_loop(0, nblk // 3, super_body, acc0, unroll=False)

    for b in (nblk - 2, nblk - 1):
        drain_in(bufs[b % 3])
        acc = compute(bufs[b % 3], acc)

    accv[...] = acc
    pltpu.sync_copy(accv, out_hbm.at[wid])


def _tc_reduce_body(part_ref, out_ref):
    out_ref[0, 0] = jnp.sum(part_ref[...])


def kernel(latent_z, latent_w, bias, analytical_i, analytical_j):
    sc_call = pl.kernel(
        _sc_likelihood,
        out_type=jax.ShapeDtypeStruct((_NW, _L), jnp.float32),
        mesh=plsc.VectorSubcoreMesh(core_axis_name="c", subcore_axis_name="s"),
        scratch_types=[
            pltpu.VMEM((_B,), jnp.int32),
            pltpu.VMEM((_B,), jnp.int32),
            pltpu.VMEM((_B, 16), jnp.float32),
            pltpu.VMEM((_B, 16), jnp.float32),
            pltpu.VMEM((_B,), jnp.int32),
            pltpu.VMEM((_B,), jnp.int32),
            pltpu.VMEM((_B, 16), jnp.float32),
            pltpu.VMEM((_B, 16), jnp.float32),
            pltpu.VMEM((_B,), jnp.int32),
            pltpu.VMEM((_B,), jnp.int32),
            pltpu.VMEM((_B, 16), jnp.float32),
            pltpu.VMEM((_B, 16), jnp.float32),
            pltpu.VMEM((_L,), jnp.float32),
            pltpu.VMEM((_L,), jnp.float32),
            pltpu.SemaphoreType.DMA,
            pltpu.SemaphoreType.DMA,
            pltpu.SemaphoreType.DMA,
            pltpu.SemaphoreType.DMA,
        ],
        compiler_params=pltpu.CompilerParams(
            needs_layout_passes=False, use_tc_tiling_on_sc=False),
    )
    partials = sc_call(latent_z, latent_w, analytical_i, analytical_j,
                       jnp.broadcast_to(bias, (_L,)))
    return jnp.sum(partials)
